# baseline (speedup 1.0000x reference)
import functools

import jax
import jax.numpy as jnp
from jax import lax
from jax.experimental import pallas as pl
from jax.experimental.pallas import tpu as pltpu
from jax.experimental.pallas import tpu_sc as plsc

N = 10000
E = 320000
L = 3
H = 128
FF = 512
NG = 64
NA = 5
R = 2000
NB = N // R
NEG = -1e30

SC_NC = 2
SC_NS = 16
SC_NW = SC_NC * SC_NS
EPW = E // SC_NW
CH = 40
NCH = EPW // CH
NP = 10240
RPS = NP // SC_NS


def _ln(t, g, b, eps=1e-5):
    m = jnp.mean(t, axis=-1, keepdims=True)
    v = jnp.mean((t - m) ** 2, axis=-1, keepdims=True)
    return (t - m) * lax.rsqrt(v + eps) * g + b


def _gelu(t):
    return 0.5 * t * (1.0 + lax.erf(t * (2.0 ** -0.5)))


def _dotT(a, b):
    return lax.dot_general(a, b, (((1,), (1,)), ((), ())),
                           preferred_element_type=jnp.float32)


def _dot0(a, b):
    return lax.dot_general(a, b, (((0,), (0,)), ((), ())),
                           preferred_element_type=jnp.float32)


def _onehot_t(ids, ncls):
    return (lax.broadcasted_iota(jnp.int32, (ncls, 1), 0) == ids
            ).astype(jnp.float32)


def _write_stage_a(hl, bond_ref, hl_ref, g_ref):
    hl_ref[...] = hl
    for a in range(NA):
        g_ref[a] = _gelu(hl + bond_ref[a:a + 1, :])


def _node_mlp(hl, ag_ref, eps_ref, w1_ref, b1_ref, lg_ref, lb_ref, w2_ref,
              b2_ref, ng_ref, nb_ref, last):
    t = (1.0 + eps_ref[0, 0]) * hl + ag_ref[0] + ag_ref[1]
    t = _dotT(t, w1_ref[...]) + b1_ref[...]
    t = _ln(t, lg_ref[...], lb_ref[...])
    t = _gelu(t)
    t = _dotT(t, w2_ref[...]) + b2_ref[...]
    h = _ln(t, ng_ref[...], nb_ref[...])
    if not last:
        h = _gelu(h)
    return h + hl


def _k1_body(x_ref, tab_ref, b_ref, vn_ref, bond_ref, hl_ref, g_ref):
    ids_x = x_ref[...].reshape(1, R)
    ids_b = b_ref[...].reshape(1, R)
    ohx = _onehot_t(ids_x, 128)
    oht = _onehot_t(ids_b, NG)
    hl = _dot0(ohx, tab_ref[...]) + _dot0(oht, vn_ref[...])
    _write_stage_a(hl, bond_ref, hl_ref, g_ref)


def _k1(x3, atom_pad, batch3, vn0, bond):
    return pl.pallas_call(
        _k1_body,
        grid=(NB,),
        in_specs=[pl.BlockSpec((1, 1, R), lambda i: (i, 0, 0)),
                  pl.BlockSpec((128, H), lambda i: (0, 0)),
                  pl.BlockSpec((1, 1, R), lambda i: (i, 0, 0)),
                  pl.BlockSpec((NG, H), lambda i: (0, 0)),
                  pl.BlockSpec((NA, H), lambda i: (0, 0))],
        out_specs=[pl.BlockSpec((R, H), lambda i: (i, 0)),
                   pl.BlockSpec((NA, R, H), lambda i: (0, i, 0))],
        out_shape=[jax.ShapeDtypeStruct((N, H), jnp.float32),
                   jax.ShapeDtypeStruct((NA, N, H), jnp.float32)],
    )(x3, atom_pad, batch3, vn0, bond)


def _vpool_body(hl_ref, b_ref, eye_ref, vp_ref):
    ids_b = b_ref[...].reshape(1, R)
    oht = _onehot_t(ids_b, NG)
    oh = _dot0(oht, eye_ref[...])
    pen = (oh - 1.0) * 1e30
    hl = hl_ref[...]

    @pl.when(pl.program_id(0) == 0)
    def _():
        vp_ref[...] = jnp.full((NG, H), NEG, jnp.float32)

    for g in range(NG):
        cand = jnp.max(hl + pen[:, g:g + 1], axis=0, keepdims=True)
        vp_ref[g:g + 1, :] = jnp.maximum(vp_ref[g:g + 1, :], cand)


def _vpool(hl, batch3, eye):
    return pl.pallas_call(
        _vpool_body,
        grid=(NB,),
        in_specs=[pl.BlockSpec((R, H), lambda i: (i, 0)),
                  pl.BlockSpec((1, 1, R), lambda i: (i, 0, 0)),
                  pl.BlockSpec((NG, NG), lambda i: (0, 0))],
        out_specs=pl.BlockSpec((NG, H), lambda i: (0, 0)),
        out_shape=jax.ShapeDtypeStruct((NG, H), jnp.float32),
    )(hl, batch3, eye)


def _k2_body(hl_ref, ag_ref, eps_ref, w1_ref, b1_ref, lg_ref, lb_ref, w2_ref,
             b2_ref, ng_ref, nb_ref, vp_ref, vn_ref, vw1_ref, vb1_ref,
             vlg_ref, vlb_ref, vw2_ref, vb2_ref, b_ref, bond_ref,
             hl2_ref, g_ref, vno_ref, vn_sc):
    @pl.when(pl.program_id(0) == 0)
    def _():
        u = _dotT(vp_ref[...], vw1_ref[...]) + vb1_ref[...]
        u = _ln(u, vlg_ref[...], vlb_ref[...])
        u = _gelu(u)
        u = _dotT(u, vw2_ref[...]) + vb2_ref[...]
        vn_new = vn_ref[...] + u
        vn_sc[...] = vn_new
        vno_ref[...] = vn_new

    h = _node_mlp(hl_ref[...], ag_ref, eps_ref, w1_ref, b1_ref, lg_ref,
                  lb_ref, w2_ref, b2_ref, ng_ref, nb_ref, last=False)
    ids_b = b_ref[...].reshape(1, R)
    oht = _onehot_t(ids_b, NG)
    hl2 = h + _dot0(oht, vn_sc[...])
    _write_stage_a(hl2, bond_ref, hl2_ref, g_ref)


def _k2(hl, aggr2, eps_l, cw, vpool, vn, vw, batch3, bond):
    full = lambda s: pl.BlockSpec(s, lambda i: tuple(0 for _ in s))
    return pl.pallas_call(
        _k2_body,
        grid=(NB,),
        in_specs=[pl.BlockSpec((R, H), lambda i: (i, 0)),
                  pl.BlockSpec((SC_NC, R, H), lambda i: (0, i, 0)),
                  full((1, 1)),
                  full((FF, H)), full((1, FF)), full((1, FF)), full((1, FF)),
                  full((H, FF)), full((1, H)), full((1, H)), full((1, H)),
                  full((NG, H)), full((NG, H)),
                  full((FF, H)), full((1, FF)), full((1, FF)), full((1, FF)),
                  full((H, FF)), full((1, H)),
                  pl.BlockSpec((1, 1, R), lambda i: (i, 0, 0)),
                  full((NA, H))],
        out_specs=[pl.BlockSpec((R, H), lambda i: (i, 0)),
                   pl.BlockSpec((NA, R, H), lambda i: (0, i, 0)),
                   pl.BlockSpec((NG, H), lambda i: (0, 0))],
        out_shape=[jax.ShapeDtypeStruct((N, H), jnp.float32),
                   jax.ShapeDtypeStruct((NA, N, H), jnp.float32),
                   jax.ShapeDtypeStruct((NG, H), jnp.float32)],
        scratch_shapes=[pltpu.VMEM((NG, H), jnp.float32)],
    )(hl, aggr2, eps_l, *cw, vpool, vn, *vw, batch3, bond)


def _k3_body(hl_ref, ag_ref, eps_ref, w1_ref, b1_ref, lg_ref, lb_ref, w2_ref,
             b2_ref, ng_ref, nb_ref, b_ref, out_ref):
    h = _node_mlp(hl_ref[...], ag_ref, eps_ref, w1_ref, b1_ref, lg_ref,
                  lb_ref, w2_ref, b2_ref, ng_ref, nb_ref, last=True)
    ids_b = b_ref[...].reshape(1, R)
    oht = _onehot_t(ids_b, NG)

    @pl.when(pl.program_id(0) == 0)
    def _():
        out_ref[...] = jnp.zeros((NG, H), jnp.float32)

    out_ref[...] += lax.dot_general(oht, h, (((1,), (0,)), ((), ())),
                                    preferred_element_type=jnp.float32)


def _k3(hl, aggr2, eps_l, cw, batch3):
    full = lambda s: pl.BlockSpec(s, lambda i: tuple(0 for _ in s))
    return pl.pallas_call(
        _k3_body,
        grid=(NB,),
        in_specs=[pl.BlockSpec((R, H), lambda i: (i, 0)),
                  pl.BlockSpec((SC_NC, R, H), lambda i: (0, i, 0)),
                  full((1, 1)),
                  full((FF, H)), full((1, FF)), full((1, FF)), full((1, FF)),
                  full((H, FF)), full((1, H)), full((1, H)), full((1, H)),
                  pl.BlockSpec((1, 1, R), lambda i: (i, 0, 0))],
        out_specs=pl.BlockSpec((NG, H), lambda i: (0, 0)),
        out_shape=jax.ShapeDtypeStruct((NG, H), jnp.float32),
    )(hl, aggr2, eps_l, *cw, batch3)


def _edge_aggr(gtab, cidx, zeros):
    mesh = plsc.VectorSubcoreMesh(core_axis_name="c", subcore_axis_name="s")

    @functools.partial(
        pl.kernel,
        out_type=jax.ShapeDtypeStruct((SC_NC, NP, H), jnp.float32),
        mesh=mesh,
        scratch_types=[
            pltpu.VMEM((4, 2, CH), jnp.int32),
            pltpu.VMEM((4, CH, H), jnp.float32),
            pltpu.VMEM_SHARED((NP, H), jnp.float32),
            pltpu.SemaphoreType.DMA,
            pltpu.SemaphoreType.DMA,
            pltpu.SemaphoreType.DMA,
        ],
    )
    def k(gtab_hbm, cidx_hbm, zeros_hbm, out_hbm,
          idx_v, rows_v, aggr_sh, semi, semg, sems):
        cid = lax.axis_index("c")
        sid = lax.axis_index("s")
        wid = sid * SC_NC + cid
        pltpu.sync_copy(zeros_hbm.at[pl.ds(sid * RPS, RPS)],
                        aggr_sh.at[pl.ds(sid * RPS, RPS)])
        plsc.subcore_barrier()

        pltpu.sync_copy(cidx_hbm.at[wid, 0], idx_v.at[0])
        pltpu.async_copy(cidx_hbm.at[wid, 1], idx_v.at[1], semi)
        pltpu.async_copy(gtab_hbm.at[idx_v.at[0, 0]], rows_v.at[0], semg)

        def body(j, carry):
            s0 = lax.rem(j, 4)
            s1 = lax.rem(j + 1, 4)
            s2 = lax.rem(j + 2, 4)

            @pl.when(j >= 2)
            def _():
                pltpu.make_async_copy(gtab_hbm.at[idx_v.at[s2, 0]],
                                      rows_v.at[s2], sems).wait()

            @pl.when(j + 2 < NCH)
            def _():
                pltpu.async_copy(cidx_hbm.at[wid, j + 2], idx_v.at[s2], semi)

            @pl.when(j + 1 < NCH)
            def _():
                pltpu.make_async_copy(cidx_hbm.at[wid, j + 1],
                                      idx_v.at[s1], semi).wait()
                pltpu.async_copy(gtab_hbm.at[idx_v.at[s1, 0]],
                                 rows_v.at[s1], semg)

            pltpu.make_async_copy(gtab_hbm.at[idx_v.at[s0, 0]],
                                  rows_v.at[s0], semg).wait()
            pltpu.async_copy(rows_v.at[s0], aggr_sh.at[idx_v.at[s0, 1]],
                             sems, add=True)
            return carry

        lax.fori_loop(0, NCH, body, 0)
        pltpu.make_async_copy(gtab_hbm.at[idx_v.at[0, 0]], rows_v.at[0],
                              sems).wait()
        pltpu.make_async_copy(gtab_hbm.at[idx_v.at[1, 0]], rows_v.at[1],
                              sems).wait()
        plsc.subcore_barrier()
        pltpu.sync_copy(aggr_sh.at[pl.ds(sid * RPS, RPS)],
                        out_hbm.at[cid, pl.ds(sid * RPS, RPS)])

    return k(gtab, cidx, zeros)


def kernel(x, edge_index, edge_attr, batch, atom_table, vn_table, eps,
           bond_tables, conv_w1, conv_b1, conv_ln_g, conv_ln_b, conv_w2,
           conv_b2, norm_g, norm_b, vn_w1, vn_b1, vn_ln_g, vn_ln_b, vn_w2,
           vn_b2):
    f32 = jnp.float32
    x3 = x.astype(jnp.int32).reshape(NB, 1, R)
    batch3 = batch.astype(jnp.int32).reshape(NB, 1, R)
    atom_pad = jnp.concatenate(
        [atom_table.astype(f32),
         jnp.zeros((128 - atom_table.shape[0], H), f32)], axis=0)
    eye = jnp.eye(NG, dtype=f32)
    gidx = (edge_attr.astype(jnp.int32) * N
            + edge_index[0].astype(jnp.int32)).reshape(SC_NW, NCH, CH)
    didx = edge_index[1].astype(jnp.int32).reshape(SC_NW, NCH, CH)
    cidx = jnp.stack([gidx, didx], axis=2)
    zeros = jnp.zeros((NP, H), f32)
    vn0 = jnp.tile(vn_table.astype(f32), (NG, 1))

    def cw(l):
        return (conv_w1[l].astype(f32), conv_b1[l].reshape(1, FF),
                conv_ln_g[l].reshape(1, FF), conv_ln_b[l].reshape(1, FF),
                conv_w2[l].astype(f32), conv_b2[l].reshape(1, H),
                norm_g[l].reshape(1, H), norm_b[l].reshape(1, H))

    def vw(l):
        return (vn_w1[l].astype(f32), vn_b1[l].reshape(1, FF),
                vn_ln_g[l].reshape(1, FF), vn_ln_b[l].reshape(1, FF),
                vn_w2[l].astype(f32), vn_b2[l].reshape(1, H))

    ep = [eps[l].reshape(1, 1).astype(f32) for l in range(L)]

    hl, gt = _k1(x3, atom_pad, batch3, vn0, bond_tables[0].astype(f32))
    aggr2 = _edge_aggr(gt.reshape(NA * N, H), cidx, zeros)
    vpool = _vpool(hl, batch3, eye)
    hl, gt, vn = _k2(hl, aggr2, ep[0], cw(0), vpool, vn0, vw(0),
                     batch3, bond_tables[1].astype(f32))
    aggr2 = _edge_aggr(gt.reshape(NA * N, H), cidx, zeros)
    vpool = _vpool(hl, batch3, eye)
    hl, gt, _ = _k2(hl, aggr2, ep[1], cw(1), vpool, vn, vw(1),
                    batch3, bond_tables[2].astype(f32))
    aggr2 = _edge_aggr(gt.reshape(NA * N, H), cidx, zeros)
    return _k3(hl, aggr2, ep[2], cw(2), batch3)


# speedup vs baseline: 1.1786x; 1.1786x over previous
import functools

import jax
import jax.numpy as jnp
from jax import lax
from jax.experimental import pallas as pl
from jax.experimental.pallas import tpu as pltpu
from jax.experimental.pallas import tpu_sc as plsc

N = 10000
E = 320000
L = 3
H = 128
FF = 512
NG = 64
NA = 5
R = 2000
NB = N // R
NEG = -1e30

SC_NC = 2
SC_NS = 16
SC_NW = SC_NC * SC_NS
EPW = E // SC_NW
CH = 80
NCH = EPW // CH
NP = 10240
RPS = NP // SC_NS


def _ln(t, g, b, eps=1e-5):
    m = jnp.mean(t, axis=-1, keepdims=True)
    v = jnp.mean((t - m) ** 2, axis=-1, keepdims=True)
    return (t - m) * lax.rsqrt(v + eps) * g + b


def _gelu(t):
    return 0.5 * t * (1.0 + lax.erf(t * (2.0 ** -0.5)))


def _dotT(a, b):
    return lax.dot_general(a, b, (((1,), (1,)), ((), ())),
                           preferred_element_type=jnp.float32)


def _dot0(a, b):
    return lax.dot_general(a, b, (((0,), (0,)), ((), ())),
                           preferred_element_type=jnp.float32)


def _onehot_t(ids, ncls):
    return (lax.broadcasted_iota(jnp.int32, (ncls, 1), 0) == ids
            ).astype(jnp.float32)


def _write_stage_a(hl, bond_ref, hl_ref, g_ref):
    hl_ref[...] = hl
    for a in range(NA):
        g_ref[a] = _gelu(hl + bond_ref[a:a + 1, :])


def _node_mlp(hl, ag_ref, eps_ref, w1_ref, b1_ref, lg_ref, lb_ref, w2_ref,
              b2_ref, ng_ref, nb_ref, last):
    t = (1.0 + eps_ref[0, 0]) * hl + ag_ref[0] + ag_ref[1]
    t = _dotT(t, w1_ref[...]) + b1_ref[...]
    t = _ln(t, lg_ref[...], lb_ref[...])
    t = _gelu(t)
    t = _dotT(t, w2_ref[...]) + b2_ref[...]
    h = _ln(t, ng_ref[...], nb_ref[...])
    if not last:
        h = _gelu(h)
    return h + hl


def _k1_body(x_ref, tab_ref, b_ref, vn_ref, bond_ref, hl_ref, g_ref):
    ids_x = x_ref[...].reshape(1, R)
    ids_b = b_ref[...].reshape(1, R)
    ohx = _onehot_t(ids_x, 128)
    oht = _onehot_t(ids_b, NG)
    hl = _dot0(ohx, tab_ref[...]) + _dot0(oht, vn_ref[...])
    _write_stage_a(hl, bond_ref, hl_ref, g_ref)


def _k1(x3, atom_pad, batch3, vn0, bond):
    return pl.pallas_call(
        _k1_body,
        grid=(NB,),
        in_specs=[pl.BlockSpec((1, 1, R), lambda i: (i, 0, 0)),
                  pl.BlockSpec((128, H), lambda i: (0, 0)),
                  pl.BlockSpec((1, 1, R), lambda i: (i, 0, 0)),
                  pl.BlockSpec((NG, H), lambda i: (0, 0)),
                  pl.BlockSpec((NA, H), lambda i: (0, 0))],
        out_specs=[pl.BlockSpec((R, H), lambda i: (i, 0)),
                   pl.BlockSpec((NA, R, H), lambda i: (0, i, 0))],
        out_shape=[jax.ShapeDtypeStruct((N, H), jnp.float32),
                   jax.ShapeDtypeStruct((NA, N, H), jnp.float32)],
    )(x3, atom_pad, batch3, vn0, bond)


def _vpool_body(hl_ref, b_ref, eye_ref, vp_ref):
    ids_b = b_ref[...].reshape(1, R)
    oht = _onehot_t(ids_b, NG)
    oh = _dot0(oht, eye_ref[...])
    pen = (oh - 1.0) * 1e30
    hl = hl_ref[...]

    @pl.when(pl.program_id(0) == 0)
    def _():
        vp_ref[...] = jnp.full((NG, H), NEG, jnp.float32)

    for g in range(NG):
        cand = jnp.max(hl + pen[:, g:g + 1], axis=0, keepdims=True)
        vp_ref[g:g + 1, :] = jnp.maximum(vp_ref[g:g + 1, :], cand)


def _vpool(hl, batch3, eye):
    return pl.pallas_call(
        _vpool_body,
        grid=(NB,),
        in_specs=[pl.BlockSpec((R, H), lambda i: (i, 0)),
                  pl.BlockSpec((1, 1, R), lambda i: (i, 0, 0)),
                  pl.BlockSpec((NG, NG), lambda i: (0, 0))],
        out_specs=pl.BlockSpec((NG, H), lambda i: (0, 0)),
        out_shape=jax.ShapeDtypeStruct((NG, H), jnp.float32),
    )(hl, batch3, eye)


def _k2_body(hl_ref, ag_ref, eps_ref, w1_ref, b1_ref, lg_ref, lb_ref, w2_ref,
             b2_ref, ng_ref, nb_ref, vp_ref, vn_ref, vw1_ref, vb1_ref,
             vlg_ref, vlb_ref, vw2_ref, vb2_ref, b_ref, bond_ref,
             hl2_ref, g_ref, vno_ref, vn_sc):
    @pl.when(pl.program_id(0) == 0)
    def _():
        u = _dotT(vp_ref[...], vw1_ref[...]) + vb1_ref[...]
        u = _ln(u, vlg_ref[...], vlb_ref[...])
        u = _gelu(u)
        u = _dotT(u, vw2_ref[...]) + vb2_ref[...]
        vn_new = vn_ref[...] + u
        vn_sc[...] = vn_new
        vno_ref[...] = vn_new

    h = _node_mlp(hl_ref[...], ag_ref, eps_ref, w1_ref, b1_ref, lg_ref,
                  lb_ref, w2_ref, b2_ref, ng_ref, nb_ref, last=False)
    ids_b = b_ref[...].reshape(1, R)
    oht = _onehot_t(ids_b, NG)
    hl2 = h + _dot0(oht, vn_sc[...])
    _write_stage_a(hl2, bond_ref, hl2_ref, g_ref)


def _k2(hl, aggr2, eps_l, cw, vpool, vn, vw, batch3, bond):
    full = lambda s: pl.BlockSpec(s, lambda i: tuple(0 for _ in s))
    return pl.pallas_call(
        _k2_body,
        grid=(NB,),
        in_specs=[pl.BlockSpec((R, H), lambda i: (i, 0)),
                  pl.BlockSpec((SC_NC, R, H), lambda i: (0, i, 0)),
                  full((1, 1)),
                  full((FF, H)), full((1, FF)), full((1, FF)), full((1, FF)),
                  full((H, FF)), full((1, H)), full((1, H)), full((1, H)),
                  full((NG, H)), full((NG, H)),
                  full((FF, H)), full((1, FF)), full((1, FF)), full((1, FF)),
                  full((H, FF)), full((1, H)),
                  pl.BlockSpec((1, 1, R), lambda i: (i, 0, 0)),
                  full((NA, H))],
        out_specs=[pl.BlockSpec((R, H), lambda i: (i, 0)),
                   pl.BlockSpec((NA, R, H), lambda i: (0, i, 0)),
                   pl.BlockSpec((NG, H), lambda i: (0, 0))],
        out_shape=[jax.ShapeDtypeStruct((N, H), jnp.float32),
                   jax.ShapeDtypeStruct((NA, N, H), jnp.float32),
                   jax.ShapeDtypeStruct((NG, H), jnp.float32)],
        scratch_shapes=[pltpu.VMEM((NG, H), jnp.float32)],
    )(hl, aggr2, eps_l, *cw, vpool, vn, *vw, batch3, bond)


def _k3_body(hl_ref, ag_ref, eps_ref, w1_ref, b1_ref, lg_ref, lb_ref, w2_ref,
             b2_ref, ng_ref, nb_ref, b_ref, out_ref):
    h = _node_mlp(hl_ref[...], ag_ref, eps_ref, w1_ref, b1_ref, lg_ref,
                  lb_ref, w2_ref, b2_ref, ng_ref, nb_ref, last=True)
    ids_b = b_ref[...].reshape(1, R)
    oht = _onehot_t(ids_b, NG)

    @pl.when(pl.program_id(0) == 0)
    def _():
        out_ref[...] = jnp.zeros((NG, H), jnp.float32)

    out_ref[...] += lax.dot_general(oht, h, (((1,), (0,)), ((), ())),
                                    preferred_element_type=jnp.float32)


def _k3(hl, aggr2, eps_l, cw, batch3):
    full = lambda s: pl.BlockSpec(s, lambda i: tuple(0 for _ in s))
    return pl.pallas_call(
        _k3_body,
        grid=(NB,),
        in_specs=[pl.BlockSpec((R, H), lambda i: (i, 0)),
                  pl.BlockSpec((SC_NC, R, H), lambda i: (0, i, 0)),
                  full((1, 1)),
                  full((FF, H)), full((1, FF)), full((1, FF)), full((1, FF)),
                  full((H, FF)), full((1, H)), full((1, H)), full((1, H)),
                  pl.BlockSpec((1, 1, R), lambda i: (i, 0, 0))],
        out_specs=pl.BlockSpec((NG, H), lambda i: (0, 0)),
        out_shape=jax.ShapeDtypeStruct((NG, H), jnp.float32),
    )(hl, aggr2, eps_l, *cw, batch3)


def _edge_aggr(gtab, cidx, zeros):
    mesh = plsc.VectorSubcoreMesh(core_axis_name="c", subcore_axis_name="s")

    @functools.partial(
        pl.kernel,
        out_type=jax.ShapeDtypeStruct((SC_NC, NP, H), jnp.float32),
        mesh=mesh,
        scratch_types=[
            pltpu.VMEM((3, 2, CH), jnp.int32),
            pltpu.VMEM((3, CH, H), jnp.float32),
            pltpu.VMEM_SHARED((NP, H), jnp.float32),
            pltpu.SemaphoreType.DMA,
            pltpu.SemaphoreType.DMA,
            pltpu.SemaphoreType.DMA,
        ],
    )
    def k(gtab_hbm, cidx_hbm, zeros_hbm, out_hbm,
          idx_v, rows_v, aggr_sh, semi, semg, sems):
        cid = lax.axis_index("c")
        sid = lax.axis_index("s")
        wid = sid * SC_NC + cid
        pltpu.sync_copy(zeros_hbm.at[pl.ds(sid * RPS, RPS)],
                        aggr_sh.at[pl.ds(sid * RPS, RPS)])
        plsc.subcore_barrier()

        pltpu.sync_copy(cidx_hbm.at[wid, 0], idx_v.at[0])
        pltpu.async_copy(cidx_hbm.at[wid, 1], idx_v.at[1], semi)
        pltpu.async_copy(gtab_hbm.at[idx_v.at[0, 0]], rows_v.at[0], semg)

        def body(j, carry):
            s0 = lax.rem(j, 3)
            s1 = lax.rem(j + 1, 3)
            s2 = lax.rem(j + 2, 3)

            @pl.when(j > 0)
            def _():
                pltpu.make_async_copy(gtab_hbm.at[idx_v.at[s0, 0]],
                                      rows_v.at[s2], sems).wait()

            @pl.when(j + 2 < NCH)
            def _():
                pltpu.async_copy(cidx_hbm.at[wid, j + 2], idx_v.at[s2], semi)

            @pl.when(j + 1 < NCH)
            def _():
                pltpu.make_async_copy(cidx_hbm.at[wid, j + 1],
                                      idx_v.at[s1], semi).wait()
                pltpu.async_copy(gtab_hbm.at[idx_v.at[s1, 0]],
                                 rows_v.at[s1], semg)

            pltpu.make_async_copy(gtab_hbm.at[idx_v.at[s0, 0]],
                                  rows_v.at[s0], semg).wait()
            pltpu.async_copy(rows_v.at[s0], aggr_sh.at[idx_v.at[s0, 1]],
                             sems, add=True)
            return carry

        lax.fori_loop(0, NCH, body, 0)
        pltpu.make_async_copy(gtab_hbm.at[idx_v.at[0, 0]], rows_v.at[0],
                              sems).wait()
        plsc.subcore_barrier()
        pltpu.sync_copy(aggr_sh.at[pl.ds(sid * RPS, RPS)],
                        out_hbm.at[cid, pl.ds(sid * RPS, RPS)])

    return k(gtab, cidx, zeros)


def kernel(x, edge_index, edge_attr, batch, atom_table, vn_table, eps,
           bond_tables, conv_w1, conv_b1, conv_ln_g, conv_ln_b, conv_w2,
           conv_b2, norm_g, norm_b, vn_w1, vn_b1, vn_ln_g, vn_ln_b, vn_w2,
           vn_b2):
    f32 = jnp.float32
    x3 = x.astype(jnp.int32).reshape(NB, 1, R)
    batch3 = batch.astype(jnp.int32).reshape(NB, 1, R)
    atom_pad = jnp.concatenate(
        [atom_table.astype(f32),
         jnp.zeros((128 - atom_table.shape[0], H), f32)], axis=0)
    eye = jnp.eye(NG, dtype=f32)
    gidx = (edge_attr.astype(jnp.int32) * N
            + edge_index[0].astype(jnp.int32)).reshape(SC_NW, NCH, CH)
    didx = edge_index[1].astype(jnp.int32).reshape(SC_NW, NCH, CH)
    cidx = jnp.stack([gidx, didx], axis=2)
    zeros = jnp.zeros((NP, H), f32)
    vn0 = jnp.tile(vn_table.astype(f32), (NG, 1))

    def cw(l):
        return (conv_w1[l].astype(f32), conv_b1[l].reshape(1, FF),
                conv_ln_g[l].reshape(1, FF), conv_ln_b[l].reshape(1, FF),
                conv_w2[l].astype(f32), conv_b2[l].reshape(1, H),
                norm_g[l].reshape(1, H), norm_b[l].reshape(1, H))

    def vw(l):
        return (vn_w1[l].astype(f32), vn_b1[l].reshape(1, FF),
                vn_ln_g[l].reshape(1, FF), vn_ln_b[l].reshape(1, FF),
                vn_w2[l].astype(f32), vn_b2[l].reshape(1, H))

    ep = [eps[l].reshape(1, 1).astype(f32) for l in range(L)]

    hl, gt = _k1(x3, atom_pad, batch3, vn0, bond_tables[0].astype(f32))
    aggr2 = _edge_aggr(gt.reshape(NA * N, H), cidx, zeros)
    vpool = _vpool(hl, batch3, eye)
    hl, gt, vn = _k2(hl, aggr2, ep[0], cw(0), vpool, vn0, vw(0),
                     batch3, bond_tables[1].astype(f32))
    aggr2 = _edge_aggr(gt.reshape(NA * N, H), cidx, zeros)
    vpool = _vpool(hl, batch3, eye)
    hl, gt, _ = _k2(hl, aggr2, ep[1], cw(1), vpool, vn, vw(1),
                    batch3, bond_tables[2].astype(f32))
    aggr2 = _edge_aggr(gt.reshape(NA * N, H), cidx, zeros)
    return _k3(hl, aggr2, ep[2], cw(2), batch3)
